# baseline (speedup 1.0000x reference)
import functools

import numpy as np

import jax
import jax.numpy as jnp
from jax.experimental import pallas as pl
from jax.experimental.pallas import tpu as pltpu
from jax.experimental.shard_map import shard_map
from jax.sharding import Mesh, PartitionSpec as P

_TEMP = 0.05
_BN = 1024


def _ce_kernel(x_ref, f_ref, t_ref, s_out, t_out, xb_ref):
    j = pl.program_id(0)
    nj = pl.num_programs(0)
    bn = f_ref.shape[0]

    @pl.when(j == 0)
    def _init():
        x = x_ref[...]
        norm = jnp.sqrt(jnp.sum(x * x, axis=1, keepdims=True))
        xn = x * ((1.0 / _TEMP) / jnp.maximum(norm, 1e-12))
        xb_ref[...] = xn.astype(jnp.bfloat16)
        s_out[...] = jnp.zeros_like(s_out)
        t_out[...] = jnp.zeros_like(t_out)

    fb = f_ref[...].astype(jnp.bfloat16)
    logits = jax.lax.dot_general(
        xb_ref[...], fb, (((1,), (1,)), ((), ())),
        preferred_element_type=jnp.float32,
    )
    s_out[...] += jnp.sum(jnp.exp(logits), axis=1, keepdims=True)
    cols = j * bn + jax.lax.broadcasted_iota(jnp.int32, logits.shape, 1)
    hit = cols == t_ref[...]
    t_out[...] += jnp.sum(jnp.where(hit, logits, 0.0), axis=1, keepdims=True)


def _partial_ce(x, f_shard, t_local, bn):
    m, k = x.shape
    n_loc = f_shard.shape[0]
    return pl.pallas_call(
        _ce_kernel,
        grid=(n_loc // bn,),
        in_specs=[
            pl.BlockSpec((m, k), lambda j: (0, 0)),
            pl.BlockSpec((bn, k), lambda j: (j, 0)),
            pl.BlockSpec((m, 1), lambda j: (0, 0)),
        ],
        out_specs=[
            pl.BlockSpec((m, 1), lambda j: (0, 0)),
            pl.BlockSpec((m, 1), lambda j: (0, 0)),
        ],
        out_shape=[
            jax.ShapeDtypeStruct((m, 1), jnp.float32),
            jax.ShapeDtypeStruct((m, 1), jnp.float32),
        ],
        scratch_shapes=[pltpu.VMEM((m, k), jnp.bfloat16)],
    )(x, f_shard, t_local)


def kernel(epoch, inputs, ema_inputs, part_out, score, targets, features,
           part_features):
    m, k = inputs.shape
    n = features.shape[0]
    devs = jax.devices()
    ndev = 2 if (len(devs) >= 2 and n % (2 * _BN) == 0) else 1
    mesh = Mesh(np.array(devs[:ndev]), ("i",))
    n_loc = n // ndev

    def sharded(x, f_shard, t):
        t_local = (t - jax.lax.axis_index("i") * n_loc).reshape(m, 1)
        s_part, t_part = _partial_ce(x, f_shard, t_local, _BN)
        s_tot = jax.lax.psum(s_part, "i")
        t_tot = jax.lax.psum(t_part, "i")
        return jnp.mean(jnp.log(s_tot) - t_tot)

    f = shard_map(
        sharded, mesh=mesh,
        in_specs=(P(), P("i", None), P()),
        out_specs=P(), check_rep=False,
    )
    return f(inputs, features, targets.astype(jnp.int32))


# speedup vs baseline: 12.0574x; 12.0574x over previous
import jax
import jax.numpy as jnp
from jax.experimental import pallas as pl
from jax.experimental.pallas import tpu as pltpu

_TEMP = 0.05
_BN = 1024


def _ce_kernel(x_ref, f_ref, t_ref, out_ref, xn_ref, s_out, t_out):
    j = pl.program_id(0)
    nj = pl.num_programs(0)
    bn = f_ref.shape[0]

    @pl.when(j == 0)
    def _init():
        x = x_ref[...]
        norm2 = jnp.sum(x * x, axis=1, keepdims=True)
        xn = x * ((1.0 / _TEMP) * jax.lax.rsqrt(norm2))
        xn_ref[...] = xn.astype(jnp.bfloat16)
        s_out[...] = jnp.zeros_like(s_out)
        t_out[...] = jnp.zeros_like(t_out)

    logits = jax.lax.dot_general(
        xn_ref[...], f_ref[...].astype(jnp.bfloat16), (((1,), (1,)), ((), ())),
        preferred_element_type=jnp.float32,
    )
    ones = jnp.ones((bn, 1), jnp.float32)
    e = jnp.exp(logits)
    s_out[...] += jax.lax.dot_general(
        e, ones, (((1,), (0,)), ((), ())),
        preferred_element_type=jnp.float32,
    )
    cols = j * bn + jax.lax.broadcasted_iota(jnp.int32, logits.shape, 1)
    masked = jnp.where(cols == t_ref[...], logits, 0.0)
    t_out[...] += jax.lax.dot_general(
        masked, ones, (((1,), (0,)), ((), ())),
        preferred_element_type=jnp.float32,
    )

    @pl.when(j == nj - 1)
    def _fin():
        per_row = jnp.log(s_out[...]) - t_out[...]
        out_ref[...] = jnp.sum(per_row, keepdims=True) * (1.0 / per_row.shape[0])


def kernel(epoch, inputs, ema_inputs, part_out, score, targets, features,
           part_features):
    m, k = inputs.shape
    n = features.shape[0]
    out = pl.pallas_call(
        _ce_kernel,
        grid=(n // _BN,),
        in_specs=[
            pl.BlockSpec((m, k), lambda j: (0, 0)),
            pl.BlockSpec((_BN, k), lambda j: (j, 0)),
            pl.BlockSpec((m, 1), lambda j: (0, 0)),
        ],
        out_specs=pl.BlockSpec((1, 1), lambda j: (0, 0)),
        out_shape=jax.ShapeDtypeStruct((1, 1), jnp.float32),
        scratch_shapes=[
            pltpu.VMEM((m, k), jnp.bfloat16),
            pltpu.VMEM((m, 1), jnp.float32),
            pltpu.VMEM((m, 1), jnp.float32),
        ],
    )(inputs, features, targets.reshape(m, 1))
    return out[0, 0]
